# Initial kernel scaffold; baseline (speedup 1.0000x reference)
import jax
import jax.numpy as jnp
from jax.experimental import pallas as pl


def kernel(z, emb):
    raise NotImplementedError("write your pallas kernel here")


# speedup vs baseline: 1.1337x; 1.1337x over previous
import functools

import jax
import jax.numpy as jnp
from jax import lax
from jax.experimental import pallas as pl
from jax.experimental.pallas import tpu as pltpu
from jax.experimental.pallas import tpu_sc as plsc

VOCAB = 8192
DIM = 32
BETA = 0.25
B = 64
PIX = 1024
N = B * PIX


_KC = 2048


def _tc_argmin_body(z_ref, emb_ref, idx_ref, loss_ref):
    zb = z_ref[0]
    em = emb_ref[...]

    zbf = zb.astype(jnp.bfloat16).astype(jnp.float32)

    zsq = zb * zb
    zn = zsq[0:1]
    for c in range(1, DIM):
        zn = zn + zsq[c : c + 1]

    e2 = em * em
    a = e2[:, 0:8] + e2[:, 8:16]
    a = a + e2[:, 16:24]
    a = a + e2[:, 24:32]
    bsum = a[:, 0:4] + a[:, 4:8]
    csum = bsum[:, 0:2] + bsum[:, 2:4]
    en = csum[:, 0:1] + csum[:, 1:2]

    half_v = []
    half_i = []
    for h in range(2):
        hv = jnp.full((PIX,), jnp.inf, jnp.float32)
        hi = jnp.zeros((PIX,), jnp.int32)
        for kk in range(VOCAB // (2 * _KC)):
            k = h * (VOCAB // (2 * _KC)) + kk
            emk = lax.slice(em, (k * _KC, 0), ((k + 1) * _KC, DIM))
            mm = lax.dot_general(
                emk,
                zbf,
                (((1,), (0,)), ((), ())),
                preferred_element_type=jnp.float32,
            )
            enk = lax.slice(en, (k * _KC, 0), ((k + 1) * _KC, 1))
            dist = (zn + enk) - 2.0 * mm
            mv = jnp.min(dist, axis=0)
            mi = jnp.argmin(dist, axis=0).astype(jnp.int32) + jnp.int32(k * _KC)
            better = mv < hv
            hv = jnp.where(better, mv, hv)
            hi = jnp.where(better, mi, hi)
        half_v.append(hv)
        half_i.append(hi)

    u = lax.bitcast_convert_type(half_v[0], jnp.uint32)
    u = (u + jnp.uint32(0x7FFF) + ((u >> 16) & jnp.uint32(1))) & jnp.uint32(0xFFFF0000)
    v0r = lax.bitcast_convert_type(u, jnp.float32)
    take1 = half_v[1] < v0r
    run_idx = jnp.where(take1, half_i[1], half_i[0])
    run_min = jnp.where(take1, half_v[1], half_v[0])

    idx_ref[0, 0, :] = run_idx
    part = jnp.sum(run_min)

    @pl.when(pl.program_id(0) == 0)
    def _init():
        loss_ref[0, 0] = 0.0

    loss_ref[0, 0] += part


def _tc_argmin_call(zr, emb):
    return pl.pallas_call(
        _tc_argmin_body,
        grid=(B,),
        in_specs=[
            pl.BlockSpec((1, DIM, PIX), lambda b: (b, 0, 0)),
            pl.BlockSpec((VOCAB, DIM), lambda b: (0, 0)),
        ],
        out_specs=[
            pl.BlockSpec((1, 1, PIX), lambda b: (b, 0, 0)),
            pl.BlockSpec(
                block_shape=(1, 1),
                index_map=lambda b: (0, 0),
                memory_space=pltpu.SMEM,
            ),
        ],
        out_shape=[
            jax.ShapeDtypeStruct((B, 1, PIX), jnp.int32),
            jax.ShapeDtypeStruct((1, 1), jnp.float32),
        ],
    )(zr, emb)


_NW = 32
_BPW = N // _NW


def _sc_gather_body(emb_hbm, idx_hbm, out_hbm, idx_v, rows_v, sem):
    wid = lax.axis_index("s") * 2 + lax.axis_index("c")
    base = wid * _BPW
    pltpu.sync_copy(idx_hbm.at[pl.ds(base, _BPW)], idx_v)
    pltpu.async_copy(emb_hbm.at[idx_v], rows_v, sem).wait()
    pltpu.sync_copy(rows_v, out_hbm.at[pl.ds(base, _BPW)])


@functools.lru_cache(maxsize=1)
def _sc_gather():
    return functools.partial(
        pl.kernel,
        out_type=jax.ShapeDtypeStruct((N, DIM), jnp.float32),
        mesh=plsc.VectorSubcoreMesh(core_axis_name="c", subcore_axis_name="s"),
        scratch_types=[
            pltpu.VMEM((_BPW,), jnp.int32),
            pltpu.VMEM((_BPW, DIM), jnp.float32),
            pltpu.SemaphoreType.DMA,
        ],
        compiler_params=pltpu.CompilerParams(use_tc_tiling_on_sc=False),
    )(_sc_gather_body)


def kernel(z, emb):
    b, c, h, w = z.shape
    zr = z.reshape(b, c, h * w)
    idx3, loss_acc = _tc_argmin_call(zr, emb)

    idx_flat = idx3.reshape(N)
    zq_flat = _sc_gather()(emb, idx_flat)

    z_q = zq_flat.reshape(b, h, w, c).transpose(0, 3, 1, 2)
    idx_out = idx3.reshape(b, h, w)
    vq_loss = (1.0 + BETA) * loss_acc[0, 0] / float(N * DIM)
    return (z_q, idx_out, vq_loss)
